# Initial kernel scaffold; baseline (speedup 1.0000x reference)
import jax
import jax.numpy as jnp
from jax.experimental import pallas as pl


def kernel(chords, table):
    raise NotImplementedError("write your pallas kernel here")


# speedup vs baseline: 101.3370x; 101.3370x over previous
import jax
import jax.numpy as jnp
from jax import lax
from jax.experimental import pallas as pl
from jax.experimental.pallas import tpu as pltpu
from jax.experimental.pallas import tpu_sc as plsc

_VOCAB = 100000
_EMBED = 64
_BATCH = 16384
_CHORD = 4

_NC, _NS = 2, 16
_NW = _NC * _NS
_CPW = _BATCH // _NW
_CHUNK = 128
_NCHUNK = _CPW // _CHUNK
_ROWS = _CHUNK * _CHORD
_NDMA = _ROWS // 128
_PAIRS = ((0, 1), (0, 2), (0, 3), (1, 2), (1, 3), (2, 3))


def _sc_body(chords_hbm, table_hbm, out_hbm,
             idx0, idx1, rows0, rows1, d0, d1, d2, d3, d4, d5,
             outst0, outst1, sem_g0, sem_g1, sem_o0, sem_o1):
    dbufs = (d0, d1, d2, d3, d4, d5)
    idx_v = (idx0, idx1)
    rows_v = (rows0, rows1)
    outst_v = (outst0, outst1)
    sem_g = (sem_g0, sem_g1)
    sem_o = (sem_o0, sem_o1)
    wid = lax.axis_index("s") * _NC + lax.axis_index("c")
    lanes = lax.iota(jnp.int32, 16)
    lane15 = lanes == 15

    def fire(t, b):
        row0 = wid * 16 + t * 4
        pltpu.sync_copy(chords_hbm.at[pl.ds(row0, _NDMA)], idx_v[b])
        descs = []
        for j in range(_NDMA):
            descs.append(pltpu.async_copy(
                table_hbm.at[idx_v[b].at[j]],
                rows_v[b].at[pl.ds(j * 128, 128)],
                sem_g[b]))
        return descs

    def compute(t, b):
        def chord_body(c, carry):
            e = []
            for i in range(_CHORD):
                e.append([rows_v[b][_CHORD * c + i, pl.ds(16 * k, 16)]
                          for k in range(4)])
            for p, (i, j) in enumerate(_PAIRS):
                acc = e[i][0] * e[j][0]
                acc = acc + e[i][1] * e[j][1]
                acc = acc + e[i][2] * e[j][2]
                acc = acc + e[i][3] * e[j][3]
                tot = plsc.cumsum(acc)
                plsc.store_compressed(dbufs[p].at[pl.ds(c, 16)], tot,
                                      mask=lane15)
            return carry

        lax.fori_loop(0, _CHUNK, chord_body, 0)

        for g in range(_CHUNK // 16):
            d = [dbufs[p][pl.ds(16 * g, 16)] for p in range(6)]
            v = []
            for i in range(_CHORD):
                f = 64 * g + 4 * lanes + i
                v.append(plsc.load_gather(idx_v[b], [f >> 7, f & 127]))
            nz = [vi != 0 for vi in v]
            m = {}
            for p, (i, j) in enumerate(_PAIRS):
                mp = (v[i] != v[j]) & nz[i] & nz[j]
                m[(i, j)] = m[(j, i)] = jnp.where(mp, d[p], 0.0)
            zero = jnp.zeros((16,), jnp.float32)
            for i in range(_CHORD):
                s = zero
                for j in range(_CHORD):
                    if j != i:
                        s = s + m[(i, j)]
                x = s * jnp.float32(1.0 / 3.0)
                neg = jnp.minimum(x, 0.0)
                u = jnp.exp(-jnp.abs(x))
                z = u / (2.0 + u)
                z2 = z * z
                poly = jnp.float32(1.0 / 9.0)
                poly = jnp.float32(1.0 / 7.0) + z2 * poly
                poly = jnp.float32(1.0 / 5.0) + z2 * poly
                poly = jnp.float32(1.0 / 3.0) + z2 * poly
                poly = jnp.float32(1.0) + z2 * poly
                res = neg - 2.0 * z * poly
                f = 64 * g + 4 * lanes + i
                plsc.store_scatter(outst_v[b], [f], res)

    gd = [None] * _NCHUNK
    od = [None] * _NCHUNK
    gd[0] = fire(0, 0)
    gd[1] = fire(1, 1)
    for t in range(_NCHUNK):
        b = t % 2
        if t >= 2:
            od[t - 2].wait()
        for dsc in gd[t]:
            dsc.wait()
        compute(t, b)
        obase = wid * (_CPW * _CHORD) + t * _ROWS
        od[t] = pltpu.async_copy(outst_v[b],
                                 out_hbm.at[pl.ds(obase, _ROWS)], sem_o[b])
        if t + 2 < _NCHUNK:
            gd[t + 2] = fire(t + 2, b)
    od[_NCHUNK - 2].wait()
    od[_NCHUNK - 1].wait()


@jax.jit
def kernel(chords, table):
    chords2d = chords.reshape(_BATCH * _CHORD // 128, 128)
    mesh = plsc.VectorSubcoreMesh(core_axis_name="c", subcore_axis_name="s",
                                  num_cores=_NC, num_subcores=_NS)
    out = pl.kernel(
        _sc_body,
        out_type=jax.ShapeDtypeStruct((_BATCH * _CHORD,), jnp.float32),
        mesh=mesh,
        compiler_params=pltpu.CompilerParams(needs_layout_passes=False,
                                             use_tc_tiling_on_sc=False),
        scratch_types=(
            [pltpu.VMEM((_NDMA, 128), jnp.int32)] * 2
            + [pltpu.VMEM((_ROWS, _EMBED), jnp.float32)] * 2
            + [pltpu.VMEM((_CHUNK + 16,), jnp.float32)] * 6
            + [pltpu.VMEM((_ROWS,), jnp.float32)] * 2
            + [pltpu.SemaphoreType.DMA] * 4
        ),
    )(chords2d, table)
    return out.reshape(_BATCH * _CHORD, 1, 1)
